# baseline (speedup 1.0000x reference)
import functools

import jax
import jax.numpy as jnp
from jax import lax
from jax.experimental import pallas as pl
from jax.experimental.pallas import tpu as pltpu
from jax.experimental.pallas import tpu_sc as plsc

_S = 512
_C = 1000
_B = 16384
_NW = 32
_RB = 32
_NBLK = _B // (_NW * _RB)


def _weights_tc_kernel(seg_ref, w_ref):
    seg = seg_ref[:]
    iota = lax.broadcasted_iota(jnp.int32, (_C, _S), 1)
    onehot = (seg == iota).astype(jnp.float32)
    counts = jnp.sum(onehot, axis=0, keepdims=True)
    recip = 1.0 / jnp.maximum(counts, 1.0)
    w_ref[:] = jnp.sum(onehot * recip, axis=1, keepdims=True)


def _column_weights(seg2d):
    return pl.pallas_call(
        _weights_tc_kernel,
        out_shape=jax.ShapeDtypeStruct((_C, 1), jnp.float32),
    )(seg2d)


def _sc_body(x_hbm, seg_hbm, w_hbm, out_hbm,
             segv, wv, xb0, xb1, ac0, ac1, si0, si1, so0, so1):
    wid = lax.axis_index("s") * 2 + lax.axis_index("c")
    pltpu.sync_copy(seg_hbm, segv)
    pltpu.sync_copy(w_hbm, wv)
    lanes = lax.iota(jnp.int32, 16)
    tail_mask = lanes >= 8
    zeros16 = jnp.zeros((16,), jnp.float32)
    blk0 = wid * _NBLK

    def in_copy(b, xbuf, sem):
        return pltpu.make_async_copy(
            x_hbm.at[pl.ds((blk0 + b) * _RB * _C, _RB * _C)], xbuf, sem)

    def out_copy(b, accbuf, sem):
        return pltpu.make_async_copy(
            accbuf, out_hbm.at[pl.ds((blk0 + b) * _RB * _S, _RB * _S)], sem)

    def compute(xbuf, accbuf):
        @plsc.parallel_loop(0, _RB * _S // 16, 1, unroll=8)
        def _(jz):
            accbuf[pl.ds(jz * 16, 16)] = zeros16

        @plsc.parallel_loop(0, 62, 1, unroll=2)
        def _(jc):
            off = jc * 16
            sv = segv[pl.ds(off, 16)]
            wv16 = wv[pl.ds(off, 16)]
            for r in range(_RB):
                v = xbuf[pl.ds(r * _C + off, 16)]
                plsc.addupdate_scatter(accbuf, [sv + r * _S], v * wv16)
        sv = segv[pl.ds(984, 16)]
        wv16 = wv[pl.ds(984, 16)]
        for r in range(_RB):
            v = xbuf[pl.ds(r * _C + 984, 16)]
            plsc.addupdate_scatter(accbuf, [sv + r * _S], v * wv16,
                                   mask=tail_mask)

    in_copy(0, xb0, si0).start()

    def bb_body(bb, _):
        b0 = 2 * bb
        b1 = 2 * bb + 1
        in_copy(b1, xb1, si1).start()
        in_copy(b0, xb0, si0).wait()

        @pl.when(bb > 0)
        def _():
            out_copy(b0, ac0, so0).wait()
        compute(xb0, ac0)
        out_copy(b0, ac0, so0).start()

        @pl.when(bb < _NBLK // 2 - 1)
        def _():
            in_copy(b0 + 2, xb0, si0).start()
        in_copy(b1, xb1, si1).wait()

        @pl.when(bb > 0)
        def _():
            out_copy(b1, ac1, so1).wait()
        compute(xb1, ac1)
        out_copy(b1, ac1, so1).start()
        return 0

    lax.fori_loop(0, _NBLK // 2, bb_body, 0)
    out_copy(_NBLK - 2, ac0, so0).wait()
    out_copy(_NBLK - 1, ac1, so1).wait()


_sc_segmean = pl.kernel(
    _sc_body,
    mesh=plsc.VectorSubcoreMesh(core_axis_name="c", subcore_axis_name="s"),
    out_type=jax.ShapeDtypeStruct((_B * _S,), jnp.float32),
    compiler_params=pltpu.CompilerParams(needs_layout_passes=False),
    scratch_types=[
        pltpu.VMEM((_C,), jnp.int32),
        pltpu.VMEM((_C,), jnp.float32),
        pltpu.VMEM((_RB * _C,), jnp.float32),
        pltpu.VMEM((_RB * _C,), jnp.float32),
        pltpu.VMEM((_RB * _S,), jnp.float32),
        pltpu.VMEM((_RB * _S,), jnp.float32),
        pltpu.SemaphoreType.DMA,
        pltpu.SemaphoreType.DMA,
        pltpu.SemaphoreType.DMA,
        pltpu.SemaphoreType.DMA,
    ],
)


def kernel(outputs, segment_ids, num_segments):
    seg = jnp.minimum(segment_ids.astype(jnp.int32), num_segments - 1)
    w = _column_weights(seg.reshape(_C, 1)).reshape(_C)
    flat = _sc_segmean(outputs.reshape(_B * _C), seg, w)
    return flat.reshape(_B, _S)


# speedup vs baseline: 2.6219x; 2.6219x over previous
import jax
import jax.numpy as jnp
from jax import lax
from jax.experimental import pallas as pl
from jax.experimental.pallas import tpu as pltpu
from jax.experimental.pallas import tpu_sc as plsc

_S = 512
_C = 1000
_CP = 1024
_B = 16384
_NW = 32
_CT = _CP // _NW


def _sc_wbuild_body(seg_hbm, w_hbm, segv, counts, wloc, sem):
    wid = lax.axis_index("s") * 2 + lax.axis_index("c")
    pltpu.sync_copy(seg_hbm, segv)
    lanes = lax.iota(jnp.int32, 16)
    ones16 = jnp.ones((16,), jnp.float32)
    zeros16 = jnp.zeros((16,), jnp.float32)
    tail_mask = lanes >= 8

    @plsc.parallel_loop(0, _S // 16, 1, unroll=8)
    def _(j):
        counts[pl.ds(j * 16, 16)] = zeros16

    @plsc.parallel_loop(0, 62, 1, unroll=4)
    def _(j):
        sv = segv[pl.ds(j * 16, 16)]
        plsc.addupdate_scatter(counts, [sv], ones16)

    sv = segv[pl.ds(984, 16)]
    plsc.addupdate_scatter(counts, [sv], ones16, mask=tail_mask)

    @plsc.parallel_loop(0, _S // 16, 1, unroll=4)
    def _(j):
        v = counts[pl.ds(j * 16, 16)]
        counts[pl.ds(j * 16, 16)] = 1.0 / jnp.maximum(v, 1.0)

    @plsc.parallel_loop(0, _CT * _S // 16, 1, unroll=8)
    def _(j):
        wloc[pl.ds(j * 16, 16)] = zeros16

    c0 = wid * _CT
    for j in range(_CT // 16):
        cidx = c0 + j * 16 + lanes
        live = cidx < _C
        cclamp = jnp.minimum(cidx, _C - 1)
        sv = plsc.load_gather(segv, [cclamp])
        rv = plsc.load_gather(counts, [sv])
        flat = (j * 16 + lanes) * _S + sv
        plsc.store_scatter(wloc, [flat], rv, mask=live)

    pltpu.make_async_copy(
        wloc, w_hbm.at[pl.ds(c0 * _S, _CT * _S)], sem).start()
    pltpu.make_async_copy(
        wloc, w_hbm.at[pl.ds(c0 * _S, _CT * _S)], sem).wait()


_sc_wbuild = pl.kernel(
    _sc_wbuild_body,
    mesh=plsc.VectorSubcoreMesh(core_axis_name="c", subcore_axis_name="s"),
    out_type=jax.ShapeDtypeStruct((_CP * _S,), jnp.float32),
    compiler_params=pltpu.CompilerParams(needs_layout_passes=False),
    scratch_types=[
        pltpu.VMEM((_C,), jnp.int32),
        pltpu.VMEM((_S,), jnp.float32),
        pltpu.VMEM((_CT * _S,), jnp.float32),
        pltpu.SemaphoreType.DMA,
    ],
)


def _matmul_tc_kernel(w_ref, x_ref, o_ref):
    w = w_ref[:_C, :].astype(jnp.bfloat16)
    o_ref[:] = lax.dot_general(
        x_ref[:].astype(jnp.bfloat16), w,
        (((1,), (0,)), ((), ())),
        preferred_element_type=jnp.float32,
    )


def kernel(outputs, segment_ids, num_segments):
    b, c = outputs.shape
    seg = jnp.minimum(segment_ids.astype(jnp.int32), num_segments - 1)
    w = _sc_wbuild(seg).reshape(_CP, _S)
    blk = 4096
    return pl.pallas_call(
        _matmul_tc_kernel,
        grid=(b // blk,),
        in_specs=[
            pl.BlockSpec((_CP, _S), lambda i: (0, 0)),
            pl.BlockSpec((blk, c), lambda i: (i, 0)),
        ],
        out_specs=pl.BlockSpec((blk, _S), lambda i: (i, 0)),
        out_shape=jax.ShapeDtypeStruct((b, _S), jnp.float32),
        compiler_params=pltpu.CompilerParams(
            dimension_semantics=("arbitrary",),
        ),
    )(w, outputs)


# speedup vs baseline: 2.7125x; 1.0346x over previous
import jax
import jax.numpy as jnp
from jax import lax
from jax.experimental import pallas as pl
from jax.experimental.pallas import tpu as pltpu
from jax.experimental.pallas import tpu_sc as plsc

_S = 512
_C = 1000
_B = 16384


def _sc_hist_body(seg_hbm, r_hbm, segv, counts):
    wid = lax.axis_index("s") * 2 + lax.axis_index("c")

    @pl.when(wid == 0)
    def _():
        pltpu.sync_copy(seg_hbm, segv)
        lanes = lax.iota(jnp.int32, 16)
        ones16 = jnp.ones((16,), jnp.float32)
        zeros16 = jnp.zeros((16,), jnp.float32)
        tail_mask = lanes >= 8

        @plsc.parallel_loop(0, _S // 16, 1, unroll=8)
        def _(j):
            counts[pl.ds(j * 16, 16)] = zeros16

        @plsc.parallel_loop(0, 62, 1, unroll=4)
        def _(j):
            sv = segv[pl.ds(j * 16, 16)]
            plsc.addupdate_scatter(counts, [sv], ones16)

        sv = segv[pl.ds(984, 16)]
        plsc.addupdate_scatter(counts, [sv], ones16, mask=tail_mask)

        @plsc.parallel_loop(0, _S // 16, 1, unroll=4)
        def _(j):
            v = counts[pl.ds(j * 16, 16)]
            counts[pl.ds(j * 16, 16)] = 1.0 / jnp.maximum(v, 1.0)

        pltpu.sync_copy(counts, r_hbm)


_sc_recip = pl.kernel(
    _sc_hist_body,
    mesh=plsc.VectorSubcoreMesh(core_axis_name="c", subcore_axis_name="s"),
    out_type=jax.ShapeDtypeStruct((_S,), jnp.float32),
    compiler_params=pltpu.CompilerParams(needs_layout_passes=False),
    scratch_types=[
        pltpu.VMEM((_C,), jnp.int32),
        pltpu.VMEM((_S,), jnp.float32),
    ],
)


def _matmul_tc_kernel(seg_ref, r_ref, x_ref, o_ref):
    seg = seg_ref[:]
    iota = lax.broadcasted_iota(jnp.int32, (_C, _S), 1)
    onehot = (seg == iota).astype(jnp.float32)
    w = (onehot * r_ref[:]).astype(jnp.bfloat16)
    o_ref[:] = lax.dot_general(
        x_ref[:].astype(jnp.bfloat16), w,
        (((1,), (0,)), ((), ())),
        preferred_element_type=jnp.float32,
    )


def kernel(outputs, segment_ids, num_segments):
    b, c = outputs.shape
    seg = jnp.minimum(segment_ids.astype(jnp.int32), num_segments - 1)
    recip = _sc_recip(seg).reshape(1, _S)
    blk = 4096
    return pl.pallas_call(
        _matmul_tc_kernel,
        grid=(b // blk,),
        in_specs=[
            pl.BlockSpec((_C, 1), lambda i: (0, 0)),
            pl.BlockSpec((1, _S), lambda i: (0, 0)),
            pl.BlockSpec((blk, c), lambda i: (i, 0)),
        ],
        out_specs=pl.BlockSpec((blk, _S), lambda i: (i, 0)),
        out_shape=jax.ShapeDtypeStruct((b, _S), jnp.float32),
        compiler_params=pltpu.CompilerParams(
            dimension_semantics=("arbitrary",),
        ),
    )(seg.reshape(_C, 1), recip, outputs)
